# Initial kernel scaffold; baseline (speedup 1.0000x reference)
import jax
import jax.numpy as jnp
from jax.experimental import pallas as pl


def kernel(x, edge_index, W1, b1, Wmu, bmu, Wls, bls):
    raise NotImplementedError("write your pallas kernel here")


# speedup vs baseline: 32.3787x; 32.3787x over previous
import jax
import jax.numpy as jnp
from jax import lax
from jax.experimental import pallas as pl
from jax.experimental.pallas import tpu as pltpu
from jax.experimental.pallas import tpu_sc as plsc

N = 10000
E = 320000
D = 128
DH = 64
DOUT = 64

NC = 2
NS = 16
NW = NC * NS

W = 128
NWIN = 80
EPT = NWIN * W
EPAD = NW * EPT
NACC = 10240
ZPT = NACC // NS // W
OPT = NACC // NS
NBUF = 4

_mesh = plsc.VectorSubcoreMesh(core_axis_name="c", subcore_axis_name="s")
_sc_params = pltpu.CompilerParams(use_tc_tiling_on_sc=False)


def _deg_body(dstp_hbm, out_hbm, dstv, buf, acc):
    cid = lax.axis_index("c")
    sid = lax.axis_index("s")
    wid = sid * NC + cid
    pltpu.sync_copy(dstp_hbm.at[wid], dstv)
    zero = jnp.zeros((16,), jnp.float32)
    one = jnp.full((16,), 1.0, jnp.float32)
    for c in range(W // 16):
        buf[0, pl.ds(c * 16, 16)] = zero
        buf[1, pl.ds(c * 16, 16)] = one
    for z in range(ZPT):
        pltpu.sync_copy(buf.at[0], acc.at[pl.ds((sid * ZPT + z) * W, W)])
    plsc.subcore_barrier()

    def body(g, carry):
        pltpu.sync_copy(buf.at[1], acc.at[dstv.at[g]], add=True)
        return carry

    lax.fori_loop(0, NWIN, body, 0)
    plsc.subcore_barrier()
    pltpu.sync_copy(acc.at[pl.ds(sid * OPT, OPT)],
                    out_hbm.at[cid, pl.ds(sid * OPT, OPT)])


_deg = pl.kernel(
    _deg_body,
    out_type=jax.ShapeDtypeStruct((NC, NACC), jnp.float32),
    mesh=_mesh,
    compiler_params=_sc_params,
    scratch_types=[
        pltpu.VMEM((NWIN, W), jnp.int32),
        pltpu.VMEM((2, W), jnp.float32),
        pltpu.VMEM_SHARED((NACC,), jnp.float32),
    ],
)


def _prop_body(ya_hbm, yb_hbm, srcp_hbm, dstp_hbm, out_hbm, srcv, dstv, rows,
               acc, s0, s1, s2, s3):
    sems = (s0, s1, s2, s3)
    cid = lax.axis_index("c")
    sid = lax.axis_index("s")
    wid = sid * NC + cid
    pltpu.sync_copy(srcp_hbm.at[wid], srcv)
    pltpu.sync_copy(dstp_hbm.at[wid], dstv)
    zero = jnp.zeros((16,), jnp.float32)

    for h, y_hbm in ((0, ya_hbm), (1, yb_hbm)):
        def zbody(r, carry):
            for c in range(DH // 16):
                rows[0, r, pl.ds(c * 16, 16)] = zero
            return carry

        lax.fori_loop(0, W, zbody, 0)
        for z in range(ZPT):
            pltpu.sync_copy(rows.at[0], acc.at[pl.ds((sid * ZPT + z) * W, W)])
        plsc.subcore_barrier()

        for b in range(NBUF):
            pltpu.async_copy(y_hbm.at[srcv.at[b]], rows.at[b], sems[b])

        def outer(o, carry):
            for b in range(NBUF):
                g = o * NBUF + b
                pltpu.make_async_copy(y_hbm.at[srcv.at[g]], rows.at[b],
                                      sems[b]).wait()
                pltpu.sync_copy(rows.at[b], acc.at[dstv.at[g]], add=True)
                gn = g + NBUF

                @pl.when(gn < NWIN)
                def _start():
                    pltpu.async_copy(y_hbm.at[srcv.at[gn]], rows.at[b],
                                     sems[b])
            return carry

        lax.fori_loop(0, NWIN // NBUF, outer, 0)
        plsc.subcore_barrier()
        pltpu.sync_copy(acc.at[pl.ds(sid * OPT, OPT)],
                        out_hbm.at[cid, h, pl.ds(sid * OPT, OPT)])


_prop = pl.kernel(
    _prop_body,
    out_type=jax.ShapeDtypeStruct((NC, 2, NACC, DH), jnp.float32),
    mesh=_mesh,
    compiler_params=_sc_params,
    scratch_types=[
        pltpu.VMEM((NWIN, W), jnp.int32),
        pltpu.VMEM((NWIN, W), jnp.int32),
        pltpu.VMEM((NBUF, W, DH), jnp.float32),
        pltpu.VMEM_SHARED((NACC, DH), jnp.float32),
        pltpu.SemaphoreType.DMA,
        pltpu.SemaphoreType.DMA,
        pltpu.SemaphoreType.DMA,
        pltpu.SemaphoreType.DMA,
    ],
)

BR = 2000


def _tcb_body(p0_ref, p1_ref, x_ref, wa_ref, wb_ref, ya_ref, yb_ref,
              dinv_ref):
    deg = 1.0 + p0_ref[...] + p1_ref[...]
    dinv = lax.rsqrt(deg)
    dinv_ref[...] = dinv
    x = x_ref[...]
    ya_ref[...] = jnp.dot(x, wa_ref[...],
                          preferred_element_type=jnp.float32) * dinv
    yb_ref[...] = jnp.dot(x, wb_ref[...],
                          preferred_element_type=jnp.float32) * dinv


_tcb = pl.pallas_call(
    _tcb_body,
    grid=(N // BR,),
    in_specs=[
        pl.BlockSpec((BR, 1), lambda i: (i, 0)),
        pl.BlockSpec((BR, 1), lambda i: (i, 0)),
        pl.BlockSpec((BR, D), lambda i: (i, 0)),
        pl.BlockSpec((D, DH), lambda i: (0, 0)),
        pl.BlockSpec((D, DH), lambda i: (0, 0)),
    ],
    out_specs=[
        pl.BlockSpec((BR, DH), lambda i: (i, 0)),
        pl.BlockSpec((BR, DH), lambda i: (i, 0)),
        pl.BlockSpec((BR, 1), lambda i: (i, 0)),
    ],
    out_shape=[
        jax.ShapeDtypeStruct((N, DH), jnp.float32),
        jax.ShapeDtypeStruct((N, DH), jnp.float32),
        jax.ShapeDtypeStruct((N, 1), jnp.float32),
    ],
)


def _tcd_body(q0a_ref, q0b_ref, q1a_ref, q1b_ref, ya_ref, yb_ref, dinv_ref,
              b1a_ref, b1b_ref, waa_ref, wab_ref, wba_ref, wbb_ref,
              y2a_ref, y2b_ref):
    dinv = dinv_ref[...]
    ha = dinv * (q0a_ref[...] + q1a_ref[...] + ya_ref[...]) + b1a_ref[...]
    hb = dinv * (q0b_ref[...] + q1b_ref[...] + yb_ref[...]) + b1b_ref[...]
    ha = jnp.maximum(ha, 0.0)
    hb = jnp.maximum(hb, 0.0)
    f32 = jnp.float32
    y2a_ref[...] = (jnp.dot(ha, waa_ref[...], preferred_element_type=f32)
                    + jnp.dot(hb, wba_ref[...], preferred_element_type=f32)
                    ) * dinv
    y2b_ref[...] = (jnp.dot(ha, wab_ref[...], preferred_element_type=f32)
                    + jnp.dot(hb, wbb_ref[...], preferred_element_type=f32)
                    ) * dinv


_tcd = pl.pallas_call(
    _tcd_body,
    grid=(N // BR,),
    in_specs=[pl.BlockSpec((BR, DH), lambda i: (i, 0))] * 6
    + [pl.BlockSpec((BR, 1), lambda i: (i, 0))]
    + [pl.BlockSpec((1, DH), lambda i: (0, 0))] * 2
    + [pl.BlockSpec((DH, DH), lambda i: (0, 0))] * 4,
    out_specs=[
        pl.BlockSpec((BR, DH), lambda i: (i, 0)),
        pl.BlockSpec((BR, DH), lambda i: (i, 0)),
    ],
    out_shape=[
        jax.ShapeDtypeStruct((N, DH), jnp.float32),
        jax.ShapeDtypeStruct((N, DH), jnp.float32),
    ],
)


def _tce_body(r0a_ref, r0b_ref, r1a_ref, r1b_ref, y2a_ref, y2b_ref, dinv_ref,
              bmu_ref, bls_ref, mu_ref, ls_ref):
    dinv = dinv_ref[...]
    mu_ref[...] = dinv * (r0a_ref[...] + r1a_ref[...] + y2a_ref[...]) \
        + bmu_ref[...]
    ls_ref[...] = dinv * (r0b_ref[...] + r1b_ref[...] + y2b_ref[...]) \
        + bls_ref[...]


_tce = pl.pallas_call(
    _tce_body,
    grid=(N // BR,),
    in_specs=[pl.BlockSpec((BR, DH), lambda i: (i, 0))] * 6
    + [pl.BlockSpec((BR, 1), lambda i: (i, 0))]
    + [pl.BlockSpec((1, DH), lambda i: (0, 0))] * 2,
    out_specs=[
        pl.BlockSpec((BR, DH), lambda i: (i, 0)),
        pl.BlockSpec((BR, DH), lambda i: (i, 0)),
    ],
    out_shape=[
        jax.ShapeDtypeStruct((N, DH), jnp.float32),
        jax.ShapeDtypeStruct((N, DH), jnp.float32),
    ],
)


def kernel(x, edge_index, W1, b1, Wmu, bmu, Wls, bls):
    src = edge_index[0]
    dst = edge_index[1]
    pad = EPAD - E
    ar = jnp.arange(pad, dtype=jnp.int32)
    psrc = jnp.concatenate([src, (ar * 7919) % N])
    pdst = jnp.concatenate([dst, N + (ar % (NACC - N))])
    srcp = psrc.reshape(NW, NWIN, W)
    dstp = pdst.reshape(NW, NWIN, W)

    degp = _deg(dstp)
    p0 = degp[0, :N, None]
    p1 = degp[1, :N, None]
    y1a, y1b, dinv = _tcb(p0, p1, x, W1[:, :DH], W1[:, DH:])

    part1 = _prop(y1a, y1b, srcp, dstp)
    y2a, y2b = _tcd(
        part1[0, 0, :N], part1[0, 1, :N], part1[1, 0, :N], part1[1, 1, :N],
        y1a, y1b, dinv, b1[None, :DH], b1[None, DH:],
        Wmu[:DH], Wls[:DH], Wmu[DH:], Wls[DH:])

    part2 = _prop(y2a, y2b, srcp, dstp)
    mu, ls = _tce(
        part2[0, 0, :N], part2[0, 1, :N], part2[1, 0, :N], part2[1, 1, :N],
        y2a, y2b, dinv, bmu[None, :], bls[None, :])
    return mu, ls


# speedup vs baseline: 35.2798x; 1.0896x over previous
import jax
import jax.numpy as jnp
from jax import lax
from jax.experimental import pallas as pl
from jax.experimental.pallas import tpu as pltpu
from jax.experimental.pallas import tpu_sc as plsc

N = 10000
E = 320000
D = 128
DH = 64
DOUT = 64

NC = 2
NS = 16
NW = NC * NS

W = 128
EP = 327680
TOTWIN = EP // W
NWIN = TOTWIN // NW
NACC = 10240
ZPT = NACC // NS // W
OPT = NACC // NS
NBUF = 4

_mesh = plsc.VectorSubcoreMesh(core_axis_name="c", subcore_axis_name="s")
_sc_params = pltpu.CompilerParams(use_tc_tiling_on_sc=False)


def _tile_id():
    cid = lax.axis_index("c")
    sid = lax.axis_index("s")
    wid = sid * NC + cid
    return cid, sid, wid, wid * NWIN


def _deg_body(ei_hbm, out_hbm, dstv, buf, acc):
    cid, sid, wid, w0 = _tile_id()
    pltpu.sync_copy(ei_hbm.at[1, pl.ds(w0, NWIN)], dstv)

    zero = jnp.zeros((16,), jnp.float32)
    one = jnp.full((16,), 1.0, jnp.float32)
    for c in range(W // 16):
        buf[0, pl.ds(c * 16, 16)] = zero
        buf[1, pl.ds(c * 16, 16)] = one
    for z in range(ZPT):
        pltpu.sync_copy(buf.at[0], acc.at[pl.ds((sid * ZPT + z) * W, W)])
    plsc.subcore_barrier()

    def body(g, carry):
        pltpu.sync_copy(buf.at[1], acc.at[dstv.at[g]], add=True)
        return carry

    lax.fori_loop(0, NWIN, body, 0)

    plsc.subcore_barrier()
    pltpu.sync_copy(acc.at[pl.ds(sid * OPT, OPT)],
                    out_hbm.at[cid, pl.ds(sid * OPT, OPT)])


_deg = pl.kernel(
    _deg_body,
    out_type=jax.ShapeDtypeStruct((NC, NACC), jnp.float32),
    mesh=_mesh,
    compiler_params=_sc_params,
    scratch_types=[
        pltpu.VMEM((NWIN, W), jnp.int32),
        pltpu.VMEM((2, W), jnp.float32),
        pltpu.VMEM_SHARED((NACC,), jnp.float32),
    ],
)


def _prop_body(ya_hbm, yb_hbm, ei_hbm, out_hbm, srcv, dstv, rows, acc,
               s0, s1, s2, s3):
    sems = (s0, s1, s2, s3)
    cid, sid, wid, w0 = _tile_id()
    pltpu.sync_copy(ei_hbm.at[0, pl.ds(w0, NWIN)], srcv)
    pltpu.sync_copy(ei_hbm.at[1, pl.ds(w0, NWIN)], dstv)

    zero = jnp.zeros((16,), jnp.float32)

    for h, y_hbm in ((0, ya_hbm), (1, yb_hbm)):
        def zbody(r, carry):
            for c in range(DH // 16):
                rows[0, r, pl.ds(c * 16, 16)] = zero
            return carry

        lax.fori_loop(0, W, zbody, 0)
        for z in range(ZPT):
            pltpu.sync_copy(rows.at[0], acc.at[pl.ds((sid * ZPT + z) * W, W)])
        plsc.subcore_barrier()

        for b in range(NBUF):
            pltpu.async_copy(y_hbm.at[srcv.at[b]], rows.at[b], sems[b])

        def outer(o, carry):
            for b in range(NBUF):
                g = o * NBUF + b
                pltpu.make_async_copy(y_hbm.at[srcv.at[g]], rows.at[b],
                                      sems[b]).wait()
                pltpu.sync_copy(rows.at[b], acc.at[dstv.at[g]], add=True)
                gn = g + NBUF

                @pl.when(gn < NWIN)
                def _start():
                    pltpu.async_copy(y_hbm.at[srcv.at[gn]], rows.at[b],
                                     sems[b])
            return carry

        lax.fori_loop(0, NWIN // NBUF, outer, 0)

        plsc.subcore_barrier()
        pltpu.sync_copy(acc.at[pl.ds(sid * OPT, OPT)],
                        out_hbm.at[cid, h, pl.ds(sid * OPT, OPT)])


_prop = pl.kernel(
    _prop_body,
    out_type=jax.ShapeDtypeStruct((NC, 2, NACC, DH), jnp.float32),
    mesh=_mesh,
    compiler_params=_sc_params,
    scratch_types=[
        pltpu.VMEM((NWIN, W), jnp.int32),
        pltpu.VMEM((NWIN, W), jnp.int32),
        pltpu.VMEM((NBUF, W, DH), jnp.float32),
        pltpu.VMEM_SHARED((NACC, DH), jnp.float32),
        pltpu.SemaphoreType.DMA,
        pltpu.SemaphoreType.DMA,
        pltpu.SemaphoreType.DMA,
        pltpu.SemaphoreType.DMA,
    ],
)

BR = 2000


def _tcb_body(p0_ref, p1_ref, x_ref, wa_ref, wb_ref, ya_ref, yb_ref,
              dinv_ref):
    p0 = p0_ref[...].reshape(BR, 1)
    p1 = p1_ref[...].reshape(BR, 1)
    deg = 1.0 + p0 + p1
    dinv = lax.rsqrt(deg)
    dinv_ref[...] = dinv
    x = x_ref[...]
    ya_ref[...] = jnp.dot(x, wa_ref[...].reshape(D, DH),
                          preferred_element_type=jnp.float32) * dinv
    yb_ref[...] = jnp.dot(x, wb_ref[...].reshape(D, DH),
                          preferred_element_type=jnp.float32) * dinv


_tcb = pl.pallas_call(
    _tcb_body,
    grid=(N // BR,),
    in_specs=[
        pl.BlockSpec((1, BR, 1), lambda i: (0, i, 0)),
        pl.BlockSpec((1, BR, 1), lambda i: (1, i, 0)),
        pl.BlockSpec((BR, D), lambda i: (i, 0)),
        pl.BlockSpec((1, D, DH), lambda i: (0, 0, 0)),
        pl.BlockSpec((1, D, DH), lambda i: (1, 0, 0)),
    ],
    out_specs=[
        pl.BlockSpec((BR, DH), lambda i: (i, 0)),
        pl.BlockSpec((BR, DH), lambda i: (i, 0)),
        pl.BlockSpec((BR, 1), lambda i: (i, 0)),
    ],
    out_shape=[
        jax.ShapeDtypeStruct((N, DH), jnp.float32),
        jax.ShapeDtypeStruct((N, DH), jnp.float32),
        jax.ShapeDtypeStruct((N, 1), jnp.float32),
    ],
)


def _part_spec(c, h):
    return pl.BlockSpec((1, 1, BR, DH), lambda i, c=c, h=h: (c, h, i, 0))


def _tcd_body(q0a_ref, q0b_ref, q1a_ref, q1b_ref, ya_ref, yb_ref, dinv_ref,
              b1_ref, waa_ref, wab_ref, wba_ref, wbb_ref,
              y2a_ref, y2b_ref):
    dinv = dinv_ref[...]
    q0a = q0a_ref[...].reshape(BR, DH)
    q0b = q0b_ref[...].reshape(BR, DH)
    q1a = q1a_ref[...].reshape(BR, DH)
    q1b = q1b_ref[...].reshape(BR, DH)
    b1a = b1_ref[:, :DH]
    b1b = b1_ref[:, DH:]
    ha = dinv * (q0a + q1a + ya_ref[...]) + b1a
    hb = dinv * (q0b + q1b + yb_ref[...]) + b1b
    ha = jnp.maximum(ha, 0.0)
    hb = jnp.maximum(hb, 0.0)
    f32 = jnp.float32
    waa = waa_ref[...]
    wab = wab_ref[...]
    wba = wba_ref[...]
    wbb = wbb_ref[...]
    y2a_ref[...] = (jnp.dot(ha, waa, preferred_element_type=f32)
                    + jnp.dot(hb, wba, preferred_element_type=f32)) * dinv
    y2b_ref[...] = (jnp.dot(ha, wab, preferred_element_type=f32)
                    + jnp.dot(hb, wbb, preferred_element_type=f32)) * dinv


_tcd = pl.pallas_call(
    _tcd_body,
    grid=(N // BR,),
    in_specs=[_part_spec(0, 0), _part_spec(0, 1),
              _part_spec(1, 0), _part_spec(1, 1)]
    + [pl.BlockSpec((BR, DH), lambda i: (i, 0))] * 2
    + [pl.BlockSpec((BR, 1), lambda i: (i, 0)),
       pl.BlockSpec((1, D), lambda i: (0, 0)),
       pl.BlockSpec((DH, DH), lambda i: (0, 0)),
       pl.BlockSpec((DH, DH), lambda i: (0, 0)),
       pl.BlockSpec((DH, DH), lambda i: (1, 0)),
       pl.BlockSpec((DH, DH), lambda i: (1, 0))],
    out_specs=[
        pl.BlockSpec((BR, DH), lambda i: (i, 0)),
        pl.BlockSpec((BR, DH), lambda i: (i, 0)),
    ],
    out_shape=[
        jax.ShapeDtypeStruct((N, DH), jnp.float32),
        jax.ShapeDtypeStruct((N, DH), jnp.float32),
    ],
)


def _tce_body(r0a_ref, r0b_ref, r1a_ref, r1b_ref, y2a_ref, y2b_ref, dinv_ref,
              bmu_ref, bls_ref, mu_ref, ls_ref):
    dinv = dinv_ref[...]
    r0a = r0a_ref[...].reshape(BR, DH)
    r0b = r0b_ref[...].reshape(BR, DH)
    r1a = r1a_ref[...].reshape(BR, DH)
    r1b = r1b_ref[...].reshape(BR, DH)
    bmu = bmu_ref[...].reshape(1, DH)
    bls = bls_ref[...].reshape(1, DH)
    mu_ref[...] = dinv * (r0a + r1a + y2a_ref[...]) + bmu
    ls_ref[...] = dinv * (r0b + r1b + y2b_ref[...]) + bls


_tce = pl.pallas_call(
    _tce_body,
    grid=(N // BR,),
    in_specs=[_part_spec(0, 0), _part_spec(0, 1),
              _part_spec(1, 0), _part_spec(1, 1)]
    + [pl.BlockSpec((BR, DH), lambda i: (i, 0))] * 2
    + [pl.BlockSpec((BR, 1), lambda i: (i, 0)),
       pl.BlockSpec((1, DH), lambda i: (0, 0)),
       pl.BlockSpec((1, DH), lambda i: (0, 0))],
    out_specs=[
        pl.BlockSpec((BR, DH), lambda i: (i, 0)),
        pl.BlockSpec((BR, DH), lambda i: (i, 0)),
    ],
    out_shape=[
        jax.ShapeDtypeStruct((N, DH), jnp.float32),
        jax.ShapeDtypeStruct((N, DH), jnp.float32),
    ],
)


def kernel(x, edge_index, W1, b1, Wmu, bmu, Wls, bls):
    pad = EP - E
    idx = jnp.arange(pad, dtype=edge_index.dtype)
    src_pad = idx % N
    dst_pad = N + idx % (NACC - N)
    ei = jnp.concatenate(
        [edge_index,
         jnp.stack([src_pad, dst_pad])], axis=1)
    ei3 = ei.reshape(2, TOTWIN, W)

    degp = _deg(ei3)
    degr = degp.reshape(NC, NACC, 1)
    w1h = W1.reshape(D, 2, DH).transpose(1, 0, 2)
    y1a, y1b, dinv = _tcb(degr, degr, x, w1h, w1h)

    part1 = _prop(y1a, y1b, ei3)
    y2a, y2b = _tcd(part1, part1, part1, part1, y1a, y1b, dinv,
                    b1.reshape(1, D), Wmu, Wls, Wmu, Wls)

    part2 = _prop(y2a, y2b, ei3)
    mu, ls = _tce(part2, part2, part2, part2, y2a, y2b, dinv,
                  bmu.reshape(1, DOUT), bls.reshape(1, DOUT))
    return mu, ls


# speedup vs baseline: 39.7066x; 1.1255x over previous
import jax
import jax.numpy as jnp
from jax import lax
from jax.experimental import pallas as pl
from jax.experimental.pallas import tpu as pltpu
from jax.experimental.pallas import tpu_sc as plsc

N = 10000
E = 320000
D = 128
DH = 64
DOUT = 64

NC = 2
NS = 16
NW = NC * NS

W = 128
EP = 327680
TOTWIN = EP // W
NWIN = TOTWIN // NW
NACC = 10240
ZPT = NACC // NS // W
OPT = NACC // NS
NBUF = 4

_mesh = plsc.VectorSubcoreMesh(core_axis_name="c", subcore_axis_name="s")
_sc_params = pltpu.CompilerParams(use_tc_tiling_on_sc=False)


def _tile_id():
    cid = lax.axis_index("c")
    sid = lax.axis_index("s")
    wid = sid * NC + cid
    return cid, sid, wid, wid * NWIN


def _deg_body(ei_hbm, out_hbm, dstv, buf, acc):
    cid, sid, wid, w0 = _tile_id()
    pltpu.sync_copy(ei_hbm.at[2, pl.ds(w0, NWIN)], dstv)

    zero = jnp.zeros((16,), jnp.float32)
    one = jnp.full((16,), 1.0, jnp.float32)
    for c in range(W // 16):
        buf[0, pl.ds(c * 16, 16)] = zero
        buf[1, pl.ds(c * 16, 16)] = one
    for z in range(ZPT):
        pltpu.sync_copy(buf.at[0], acc.at[pl.ds((sid * ZPT + z) * W, W)])
    plsc.subcore_barrier()

    def body(g, carry):
        pltpu.sync_copy(buf.at[1], acc.at[dstv.at[g]], add=True)
        return carry

    lax.fori_loop(0, NWIN, body, 0)

    plsc.subcore_barrier()
    pltpu.sync_copy(acc.at[pl.ds(sid * OPT, OPT)],
                    out_hbm.at[cid, pl.ds(sid * OPT, OPT)])


_deg = pl.kernel(
    _deg_body,
    out_type=jax.ShapeDtypeStruct((NC, NACC), jnp.float32),
    mesh=_mesh,
    compiler_params=_sc_params,
    scratch_types=[
        pltpu.VMEM((NWIN, W), jnp.int32),
        pltpu.VMEM((2, W), jnp.float32),
        pltpu.VMEM_SHARED((NACC,), jnp.float32),
    ],
)


def _prop_body(y_hbm, ei_hbm, out_hbm, srcv, dstv, rows, acc, s0, s1, s2, s3):
    sems = (s0, s1, s2, s3)
    cid, sid, wid, w0 = _tile_id()
    pltpu.sync_copy(ei_hbm.at[2, pl.ds(w0, NWIN)], dstv)

    zero = jnp.zeros((16,), jnp.float32)

    for h in (0, 1):
        pltpu.sync_copy(ei_hbm.at[h, pl.ds(w0, NWIN)], srcv)

        def zbody(r, carry):
            for c in range(DH // 16):
                rows[0, r, pl.ds(c * 16, 16)] = zero
            return carry

        lax.fori_loop(0, W, zbody, 0)
        for z in range(ZPT):
            pltpu.sync_copy(rows.at[0], acc.at[pl.ds((sid * ZPT + z) * W, W)])
        plsc.subcore_barrier()

        for b in range(NBUF):
            pltpu.async_copy(y_hbm.at[srcv.at[b]], rows.at[b], sems[b])

        def outer(o, carry):
            for b in range(NBUF):
                g = o * NBUF + b
                pltpu.make_async_copy(y_hbm.at[srcv.at[g]], rows.at[b],
                                      sems[b]).wait()
                pltpu.sync_copy(rows.at[b], acc.at[dstv.at[g]], add=True)
                gn = g + NBUF

                @pl.when(gn < NWIN)
                def _start():
                    pltpu.async_copy(y_hbm.at[srcv.at[gn]], rows.at[b],
                                     sems[b])
            return carry

        lax.fori_loop(0, NWIN // NBUF, outer, 0)

        plsc.subcore_barrier()
        pltpu.sync_copy(acc.at[pl.ds(sid * OPT, OPT)],
                        out_hbm.at[cid, pl.ds(sid * OPT, OPT),
                                   pl.ds(h * DH, DH)])


_prop = pl.kernel(
    _prop_body,
    out_type=jax.ShapeDtypeStruct((NC, NACC, D), jnp.float32),
    mesh=_mesh,
    compiler_params=_sc_params,
    scratch_types=[
        pltpu.VMEM((NWIN, W), jnp.int32),
        pltpu.VMEM((NWIN, W), jnp.int32),
        pltpu.VMEM((NBUF, W, DH), jnp.float32),
        pltpu.VMEM_SHARED((NACC, DH), jnp.float32),
        pltpu.SemaphoreType.DMA,
        pltpu.SemaphoreType.DMA,
        pltpu.SemaphoreType.DMA,
        pltpu.SemaphoreType.DMA,
    ],
)

BR = 2000


def _tcb_body(p0_ref, p1_ref, x_ref, w_ref, y_ref, dinv_ref):
    p0 = p0_ref[...].reshape(BR, 1)
    p1 = p1_ref[...].reshape(BR, 1)
    deg = 1.0 + p0 + p1
    dinv = lax.rsqrt(deg)
    dinv_ref[...] = dinv
    y_ref[...] = jnp.dot(x_ref[...], w_ref[...],
                         preferred_element_type=jnp.float32) * dinv


_tcb = pl.pallas_call(
    _tcb_body,
    grid=(N // BR,),
    in_specs=[
        pl.BlockSpec((1, BR, 1), lambda i: (0, i, 0)),
        pl.BlockSpec((1, BR, 1), lambda i: (1, i, 0)),
        pl.BlockSpec((BR, D), lambda i: (i, 0)),
        pl.BlockSpec((D, D), lambda i: (0, 0)),
    ],
    out_specs=[
        pl.BlockSpec((BR, D), lambda i: (i, 0)),
        pl.BlockSpec((BR, 1), lambda i: (i, 0)),
    ],
    out_shape=[
        jax.ShapeDtypeStruct((N, D), jnp.float32),
        jax.ShapeDtypeStruct((N, 1), jnp.float32),
    ],
)


def _part_spec(c):
    return pl.BlockSpec((1, BR, D), lambda i, c=c: (c, i, 0))


def _tcd_body(q0_ref, q1_ref, y_ref, dinv_ref, b1_ref, w_ref, y2_ref):
    dinv = dinv_ref[...]
    q0 = q0_ref[...].reshape(BR, D)
    q1 = q1_ref[...].reshape(BR, D)
    h = dinv * (q0 + q1 + y_ref[...]) + b1_ref[...]
    h = jnp.maximum(h, 0.0)
    y2_ref[...] = jnp.dot(h, w_ref[...],
                          preferred_element_type=jnp.float32) * dinv


_tcd = pl.pallas_call(
    _tcd_body,
    grid=(N // BR,),
    in_specs=[_part_spec(0), _part_spec(1),
              pl.BlockSpec((BR, D), lambda i: (i, 0)),
              pl.BlockSpec((BR, 1), lambda i: (i, 0)),
              pl.BlockSpec((1, D), lambda i: (0, 0)),
              pl.BlockSpec((D, D), lambda i: (0, 0))],
    out_specs=pl.BlockSpec((BR, D), lambda i: (i, 0)),
    out_shape=jax.ShapeDtypeStruct((N, D), jnp.float32),
)


def _tce_body(q0_ref, q1_ref, y2_ref, dinv_ref, b_ref, out_ref):
    dinv = dinv_ref[...]
    q0 = q0_ref[...].reshape(BR, D)
    q1 = q1_ref[...].reshape(BR, D)
    out_ref[...] = dinv * (q0 + q1 + y2_ref[...]) + b_ref[...]


_tce = pl.pallas_call(
    _tce_body,
    grid=(N // BR,),
    in_specs=[_part_spec(0), _part_spec(1),
              pl.BlockSpec((BR, D), lambda i: (i, 0)),
              pl.BlockSpec((BR, 1), lambda i: (i, 0)),
              pl.BlockSpec((1, D), lambda i: (0, 0))],
    out_specs=pl.BlockSpec((BR, D), lambda i: (i, 0)),
    out_shape=jax.ShapeDtypeStruct((N, D), jnp.float32),
)


def kernel(x, edge_index, W1, b1, Wmu, bmu, Wls, bls):
    pad = EP - E
    idx = jnp.arange(pad, dtype=edge_index.dtype)
    srcp = jnp.concatenate([edge_index[0], idx % N])
    dstp = jnp.concatenate([edge_index[1], N + idx % (NACC - N)])
    ei3 = jnp.stack([2 * srcp, 2 * srcp + 1, dstp]).reshape(3, TOTWIN, W)

    degp = _deg(ei3)
    degr = degp.reshape(NC, NACC, 1)
    y1, dinv = _tcb(degr, degr, x, W1)

    part1 = _prop(y1.reshape(2 * N, DH), ei3)
    wcat = jnp.concatenate([Wmu, Wls], axis=1)
    y2 = _tcd(part1, part1, y1, dinv, b1.reshape(1, D), wcat)

    part2 = _prop(y2.reshape(2 * N, DH), ei3)
    bcat = jnp.concatenate([bmu, bls]).reshape(1, D)
    out = _tce(part2, part2, y2, dinv, bcat)
    return out[:, :DOUT], out[:, DOUT:]
